# Initial kernel scaffold; baseline (speedup 1.0000x reference)
import jax
import jax.numpy as jnp
from jax.experimental import pallas as pl


def kernel(inputs_ids, input_lens, embedding, fc_w, fc_b):
    raise NotImplementedError("write your pallas kernel here")


# speedup vs baseline: 5.1467x; 5.1467x over previous
import functools

import jax
import jax.numpy as jnp
from jax import lax
from jax.experimental import pallas as pl
from jax.experimental.pallas import tpu as pltpu
from jax.experimental.pallas import tpu_sc as plsc

VOCAB = 100000
EMB = 64
NL = 9
DPAD = 16

NC = 2
NS = 16
NW = NC * NS


def _fold_body(emb_ref, w_ref, b_ref, out_ref):
    out_ref[...] = (
        jnp.dot(emb_ref[...], w_ref[...], preferred_element_type=jnp.float32)
        + b_ref[...]
    )


def _fold_table(embedding, fc_w, fc_b):
    w = jnp.zeros((EMB, DPAD), jnp.float32).at[:, :NL].set(fc_w)
    b = jnp.zeros((1, DPAD), jnp.float32).at[0, :NL].set(fc_b)
    blk = 2000
    return pl.pallas_call(
        _fold_body,
        grid=(VOCAB // blk,),
        in_specs=[
            pl.BlockSpec((blk, EMB), lambda i: (i, 0)),
            pl.BlockSpec((EMB, DPAD), lambda i: (0, 0)),
            pl.BlockSpec((1, DPAD), lambda i: (0, 0)),
        ],
        out_specs=pl.BlockSpec((blk, DPAD), lambda i: (i, 0)),
        out_shape=jax.ShapeDtypeStruct((VOCAB, DPAD), jnp.float32),
    )(embedding, w, b)


def _make_gather(ntok):
    per_w = ntok // NW
    chunk = 3200
    steps = per_w // chunk
    assert per_w % chunk == 0

    @functools.partial(
        pl.kernel,
        mesh=plsc.VectorSubcoreMesh(core_axis_name="c", subcore_axis_name="s"),
        out_type=jax.ShapeDtypeStruct((ntok, DPAD), jnp.float32),
        scratch_types=[
            pltpu.VMEM((chunk,), jnp.int32),
            pltpu.VMEM((chunk, DPAD), jnp.float32),
            pltpu.SemaphoreType.DMA,
        ],
        compiler_params=pltpu.CompilerParams(use_tc_tiling_on_sc=False),
    )
    def gather(tbl_hbm, idx_hbm, out_hbm, idx_v, rows_v, sem):
        wid = lax.axis_index("s") * NC + lax.axis_index("c")
        base = wid * per_w

        def body(i, carry):
            off = base + i * chunk
            pltpu.sync_copy(idx_hbm.at[pl.ds(off, chunk)], idx_v)
            pltpu.async_copy(tbl_hbm.at[idx_v], rows_v, sem).wait()
            pltpu.sync_copy(rows_v, out_hbm.at[pl.ds(off, chunk), :])
            return carry

        lax.fori_loop(0, steps, body, 0)

    return gather


def kernel(inputs_ids, input_lens, embedding, fc_w, fc_b):
    del input_lens
    b, l = inputs_ids.shape
    tbl = _fold_table(embedding, fc_w, fc_b)
    idx = inputs_ids.reshape(-1).astype(jnp.int32)
    out = _make_gather(b * l)(tbl, idx)
    return out[:, :NL].reshape(b, l, NL)
